# baseline (speedup 1.0000x reference)
import jax
import jax.numpy as jnp
from jax.experimental import pallas as pl

BATCH = 32
WIN = 128
ENC_IN = 64
C_OUT = 64
D_MODEL = 1024
D_FF = 512
N_MEM = 128
TOPK = 32
D_GE = 64
SHRINK = 0.0025

_F32 = jnp.float32


def _body(x_ref, eW_ref, eb_ref, Wq_ref, bq_ref, Wk_ref, bk_ref, gW_ref,
          gb_ref, mem_ref, W1_ref, b1_ref, W2_ref, b2_ref, out_ref):
    dn = lambda a, b, dims: jax.lax.dot_general(
        a, b, (dims, ((), ())), preferred_element_type=_F32)

    xb = x_ref[0]
    xe = dn(xb, eW_ref[...], ((1,), (0,))) + eb_ref[...]

    q = dn(xe, Wq_ref[...], ((0,), (0,))) + bq_ref[...]
    k = dn(xe, Wk_ref[...], ((0,), (0,))) + bk_ref[...]
    adj = dn(q, k, ((1,), (1,))) * 0.125

    r_io = jax.lax.broadcasted_iota(jnp.int32, (D_MODEL, D_MODEL), 0)
    c_io = jax.lax.broadcasted_iota(jnp.int32, (D_MODEL, D_MODEL), 1)
    adj = jnp.where(r_io == c_io, -jnp.inf, adj)

    ai = jax.lax.bitcast_convert_type(adj, jnp.int32)
    ikey = jnp.where(ai >= 0, ai, ai ^ jnp.int32(0x7FFFFFFF))

    kf = jnp.float32(TOPK)
    ones_col = jnp.ones((128, 1), _F32)

    def count_ge(t):
        m = (ikey >= t).astype(_F32)
        p = (((m[:, 0:128] + m[:, 128:256]) + (m[:, 256:384] + m[:, 384:512]))
             + ((m[:, 512:640] + m[:, 640:768])
                + (m[:, 768:896] + m[:, 896:1024])))
        return dn(p, ones_col, ((1,), (0,)))

    zero = jnp.zeros((D_MODEL, 1), jnp.int32)
    cand0 = jnp.where(count_ge(zero) >= kf, zero,
                      jnp.full((D_MODEL, 1), jnp.int32(-2147483648)))

    def vstep(i, cand):
        trial = cand | (jnp.int32(1) << (jnp.int32(30) - i))
        return jnp.where(count_ge(trial) >= kf, trial, cand)

    vk = jax.lax.fori_loop(0, 31, vstep, cand0, unroll=8)

    Af = (ikey >= vk).astype(_F32)
    no_ties = jnp.all(jnp.sum(Af, axis=1, keepdims=True) == kf)

    def _fast(_):
        return Af

    def _slow(_):
        need = kf - jnp.sum((ikey > vk).astype(_F32), axis=1, keepdims=True)
        tie = ikey == vk

        def count_tie_le(t):
            return jnp.sum((tie & (c_io <= t)).astype(_F32), axis=1,
                           keepdims=True)

        def istep(i, cand):
            trial = cand | (jnp.int32(1) << (jnp.int32(9) - i))
            return jnp.where(count_tie_le(trial) <= need, trial, cand)

        tstar = jax.lax.fori_loop(0, 10, istep,
                                  jnp.zeros((D_MODEL, 1), jnp.int32))
        return ((ikey > vk) | (tie & (c_io <= tstar))).astype(_F32)

    A = jax.lax.cond(no_ties, _fast, _slow, None)

    deg = jnp.sum(A, axis=0, keepdims=True) + 1.0
    dinv = jax.lax.rsqrt(deg)
    xwT = dn(gW_ref[...], xe, ((0,), (0,)))
    yT = xwT * dinv
    sT = dn(yT, A, ((1,), (0,)))
    hT = dinv * sT + xwT / deg + gb_ref[...]

    logits = dn(hT, mem_ref[...], ((1,), (1,)))
    attn = jax.nn.softmax(logits, axis=-1)
    attn = (jax.nn.relu(attn - SHRINK) * attn) / (jnp.abs(attn - SHRINK) + 1e-12)
    attn = attn / (jnp.sum(attn, axis=-1, keepdims=True) + 1e-12)
    read = dn(attn, mem_ref[...], ((1,), (0,)))

    W1 = W1_ref[...]
    hdec = jax.nn.gelu(dn(hT, W1[:D_MODEL], ((1,), (0,)))
                       + dn(read, W1[D_MODEL:], ((1,), (0,)))
                       + b1_ref[...])
    out_ref[0] = dn(hdec, W2_ref[...], ((1,), (0,))) + b2_ref[...]


def _full(shape):
    return pl.BlockSpec(shape, lambda b: (0,) * len(shape))


@jax.jit
def kernel(x, embed_W, embed_b, Wq, bq, Wk, bk, gcn_W, gcn_b, mem,
           dec_W1, dec_b1, dec_W2, dec_b2):
    call = pl.pallas_call(
        _body,
        grid=(BATCH,),
        in_specs=[
            pl.BlockSpec((1, WIN, ENC_IN), lambda b: (b, 0, 0)),
            _full((ENC_IN, D_MODEL)), _full((1, D_MODEL)),
            _full((WIN, D_GE)), _full((1, D_GE)),
            _full((WIN, D_GE)), _full((1, D_GE)),
            _full((WIN, WIN)), _full((WIN, 1)),
            _full((N_MEM, D_MODEL)),
            _full((2 * D_MODEL, D_FF)), _full((1, D_FF)),
            _full((D_FF, C_OUT)), _full((1, C_OUT)),
        ],
        out_specs=pl.BlockSpec((1, WIN, C_OUT), lambda b: (b, 0, 0)),
        out_shape=jax.ShapeDtypeStruct((BATCH, WIN, C_OUT), jnp.float32),
    )
    return call(x, embed_W, embed_b.reshape(1, D_MODEL),
                Wq, bq.reshape(1, D_GE), Wk, bk.reshape(1, D_GE),
                gcn_W, gcn_b.reshape(WIN, 1), mem,
                dec_W1, dec_b1.reshape(1, D_FF),
                dec_W2, dec_b2.reshape(1, C_OUT))


# speedup vs baseline: 1.0473x; 1.0473x over previous
import jax
import jax.numpy as jnp
from jax.experimental import pallas as pl

BATCH = 32
WIN = 128
ENC_IN = 64
C_OUT = 64
D_MODEL = 1024
D_FF = 512
N_MEM = 128
TOPK = 32
D_GE = 64
SHRINK = 0.0025

_F32 = jnp.float32


def _body(x_ref, eW_ref, eb_ref, Wq_ref, bq_ref, Wk_ref, bk_ref, gW_ref,
          gb_ref, mem_ref, W1_ref, b1_ref, W2_ref, b2_ref, out_ref):
    dn = lambda a, b, dims: jax.lax.dot_general(
        a, b, (dims, ((), ())), preferred_element_type=_F32)

    xb = x_ref[0]
    xe = dn(xb, eW_ref[...], ((1,), (0,))) + eb_ref[...]

    q = dn(xe, Wq_ref[...], ((0,), (0,))) + bq_ref[...]
    k = dn(xe, Wk_ref[...], ((0,), (0,))) + bk_ref[...]
    adj = dn(q, k, ((1,), (1,))) * 0.125

    r_io = jax.lax.broadcasted_iota(jnp.int32, (D_MODEL, D_MODEL), 0)
    c_io = jax.lax.broadcasted_iota(jnp.int32, (D_MODEL, D_MODEL), 1)
    adj = jnp.where(r_io == c_io, -jnp.inf, adj)

    ai = jax.lax.bitcast_convert_type(adj, jnp.int32)
    ikey = jnp.where(ai >= 0, ai, ai ^ jnp.int32(0x7FFFFFFF))

    kf = jnp.float32(TOPK)
    ones_col = jnp.ones((128, 1), _F32)

    def count_ge(t):
        m = (ikey >= t).astype(_F32)
        p = (((m[:, 0:128] + m[:, 128:256]) + (m[:, 256:384] + m[:, 384:512]))
             + ((m[:, 512:640] + m[:, 640:768])
                + (m[:, 768:896] + m[:, 896:1024])))
        return jnp.sum(p, axis=1, keepdims=True)

    zero = jnp.zeros((D_MODEL, 1), jnp.int32)
    cand0 = jnp.where(count_ge(zero) >= kf, zero,
                      jnp.full((D_MODEL, 1), jnp.int32(-2147483648)))

    def vstep(i, cand):
        trial = cand | (jnp.int32(1) << (jnp.int32(30) - i))
        return jnp.where(count_ge(trial) >= kf, trial, cand)

    vk = jax.lax.fori_loop(0, 31, vstep, cand0, unroll=8)

    Af = (ikey >= vk).astype(_F32)
    no_ties = jnp.all(jnp.sum(Af, axis=1, keepdims=True) == kf)

    def _fast(_):
        return Af

    def _slow(_):
        need = kf - jnp.sum((ikey > vk).astype(_F32), axis=1, keepdims=True)
        tie = ikey == vk

        def count_tie_le(t):
            return jnp.sum((tie & (c_io <= t)).astype(_F32), axis=1,
                           keepdims=True)

        def istep(i, cand):
            trial = cand | (jnp.int32(1) << (jnp.int32(9) - i))
            return jnp.where(count_tie_le(trial) <= need, trial, cand)

        tstar = jax.lax.fori_loop(0, 10, istep,
                                  jnp.zeros((D_MODEL, 1), jnp.int32))
        return ((ikey > vk) | (tie & (c_io <= tstar))).astype(_F32)

    A = jax.lax.cond(no_ties, _fast, _slow, None)

    deg = jnp.sum(A, axis=0, keepdims=True) + 1.0
    dinv = jax.lax.rsqrt(deg)
    xwT = dn(gW_ref[...], xe, ((0,), (0,)))
    yT = xwT * dinv
    sT = dn(yT, A, ((1,), (0,)))
    hT = dinv * sT + xwT / deg + gb_ref[...]

    logits = dn(hT, mem_ref[...], ((1,), (1,)))
    attn = jax.nn.softmax(logits, axis=-1)
    attn = (jax.nn.relu(attn - SHRINK) * attn) / (jnp.abs(attn - SHRINK) + 1e-12)
    attn = attn / (jnp.sum(attn, axis=-1, keepdims=True) + 1e-12)
    read = dn(attn, mem_ref[...], ((1,), (0,)))

    W1 = W1_ref[...]
    hdec = jax.nn.gelu(dn(hT, W1[:D_MODEL], ((1,), (0,)))
                       + dn(read, W1[D_MODEL:], ((1,), (0,)))
                       + b1_ref[...])
    out_ref[0] = dn(hdec, W2_ref[...], ((1,), (0,))) + b2_ref[...]


def _full(shape):
    return pl.BlockSpec(shape, lambda b: (0,) * len(shape))


@jax.jit
def kernel(x, embed_W, embed_b, Wq, bq, Wk, bk, gcn_W, gcn_b, mem,
           dec_W1, dec_b1, dec_W2, dec_b2):
    call = pl.pallas_call(
        _body,
        grid=(BATCH,),
        in_specs=[
            pl.BlockSpec((1, WIN, ENC_IN), lambda b: (b, 0, 0)),
            _full((ENC_IN, D_MODEL)), _full((1, D_MODEL)),
            _full((WIN, D_GE)), _full((1, D_GE)),
            _full((WIN, D_GE)), _full((1, D_GE)),
            _full((WIN, WIN)), _full((WIN, 1)),
            _full((N_MEM, D_MODEL)),
            _full((2 * D_MODEL, D_FF)), _full((1, D_FF)),
            _full((D_FF, C_OUT)), _full((1, C_OUT)),
        ],
        out_specs=pl.BlockSpec((1, WIN, C_OUT), lambda b: (b, 0, 0)),
        out_shape=jax.ShapeDtypeStruct((BATCH, WIN, C_OUT), jnp.float32),
    )
    return call(x, embed_W, embed_b.reshape(1, D_MODEL),
                Wq, bq.reshape(1, D_GE), Wk, bk.reshape(1, D_GE),
                gcn_W, gcn_b.reshape(WIN, 1), mem,
                dec_W1, dec_b1.reshape(1, D_FF),
                dec_W2, dec_b2.reshape(1, C_OUT))


# speedup vs baseline: 1.0791x; 1.0304x over previous
import jax
import jax.numpy as jnp
from jax.experimental import pallas as pl

BATCH = 32
WIN = 128
ENC_IN = 64
C_OUT = 64
D_MODEL = 1024
D_FF = 512
N_MEM = 128
TOPK = 32
D_GE = 64
SHRINK = 0.0025

_F32 = jnp.float32


def _body(x_ref, eW_ref, eb_ref, Wq_ref, bq_ref, Wk_ref, bk_ref, gW_ref,
          gb_ref, mem_ref, W1_ref, b1_ref, W2_ref, b2_ref, out_ref):
    dn = lambda a, b, dims: jax.lax.dot_general(
        a, b, (dims, ((), ())), preferred_element_type=_F32)

    xb = x_ref[0]
    xe = dn(xb, eW_ref[...], ((1,), (0,))) + eb_ref[...]

    q = dn(xe, Wq_ref[...], ((0,), (0,))) + bq_ref[...]
    k = dn(xe, Wk_ref[...], ((0,), (0,))) + bk_ref[...]
    adj = dn(q, k, ((1,), (1,))) * 0.125

    r_io = jax.lax.broadcasted_iota(jnp.int32, (D_MODEL, D_MODEL), 0)
    c_io = jax.lax.broadcasted_iota(jnp.int32, (D_MODEL, D_MODEL), 1)
    adj = jnp.where(r_io == c_io, -jnp.inf, adj)

    ai = jax.lax.bitcast_convert_type(adj, jnp.int32)
    ikey = jnp.where(ai >= 0, ai, ai ^ jnp.int32(0x7FFFFFFF))

    kf = jnp.float32(TOPK)

    def count_ge(t):
        return jnp.sum((ikey >= t).astype(_F32), axis=1, keepdims=True)

    zero = jnp.zeros((D_MODEL, 1), jnp.int32)
    cand0 = jnp.where(count_ge(zero) >= kf, zero,
                      jnp.full((D_MODEL, 1), jnp.int32(-2147483648)))

    def vstep(i, cand):
        trial = cand | (jnp.int32(1) << (jnp.int32(30) - i))
        return jnp.where(count_ge(trial) >= kf, trial, cand)

    vk = jax.lax.fori_loop(0, 31, vstep, cand0, unroll=16)

    Af = (ikey >= vk).astype(_F32)
    no_ties = jnp.all(jnp.sum(Af, axis=1, keepdims=True) == kf)

    def _fast(_):
        return Af

    def _slow(_):
        need = kf - jnp.sum((ikey > vk).astype(_F32), axis=1, keepdims=True)
        tie = ikey == vk

        def count_tie_le(t):
            return jnp.sum((tie & (c_io <= t)).astype(_F32), axis=1,
                           keepdims=True)

        def istep(i, cand):
            trial = cand | (jnp.int32(1) << (jnp.int32(9) - i))
            return jnp.where(count_tie_le(trial) <= need, trial, cand)

        tstar = jax.lax.fori_loop(0, 10, istep,
                                  jnp.zeros((D_MODEL, 1), jnp.int32))
        return ((ikey > vk) | (tie & (c_io <= tstar))).astype(_F32)

    A = jax.lax.cond(no_ties, _fast, _slow, None)

    deg = jnp.sum(A, axis=0, keepdims=True) + 1.0
    dinv = jax.lax.rsqrt(deg)
    xwT = dn(gW_ref[...], xe, ((0,), (0,)))
    yT = xwT * dinv
    sT = dn(yT, A, ((1,), (0,)))
    hT = dinv * sT + xwT / deg + gb_ref[...]

    logits = dn(hT, mem_ref[...], ((1,), (1,)))
    attn = jax.nn.softmax(logits, axis=-1)
    attn = (jax.nn.relu(attn - SHRINK) * attn) / (jnp.abs(attn - SHRINK) + 1e-12)
    attn = attn / (jnp.sum(attn, axis=-1, keepdims=True) + 1e-12)
    read = dn(attn, mem_ref[...], ((1,), (0,)))

    W1 = W1_ref[...]
    hdec = jax.nn.gelu(dn(hT, W1[:D_MODEL], ((1,), (0,)))
                       + dn(read, W1[D_MODEL:], ((1,), (0,)))
                       + b1_ref[...])
    out_ref[0] = dn(hdec, W2_ref[...], ((1,), (0,))) + b2_ref[...]


def _full(shape):
    return pl.BlockSpec(shape, lambda b: (0,) * len(shape))


@jax.jit
def kernel(x, embed_W, embed_b, Wq, bq, Wk, bk, gcn_W, gcn_b, mem,
           dec_W1, dec_b1, dec_W2, dec_b2):
    call = pl.pallas_call(
        _body,
        grid=(BATCH,),
        in_specs=[
            pl.BlockSpec((1, WIN, ENC_IN), lambda b: (b, 0, 0)),
            _full((ENC_IN, D_MODEL)), _full((1, D_MODEL)),
            _full((WIN, D_GE)), _full((1, D_GE)),
            _full((WIN, D_GE)), _full((1, D_GE)),
            _full((WIN, WIN)), _full((WIN, 1)),
            _full((N_MEM, D_MODEL)),
            _full((2 * D_MODEL, D_FF)), _full((1, D_FF)),
            _full((D_FF, C_OUT)), _full((1, C_OUT)),
        ],
        out_specs=pl.BlockSpec((1, WIN, C_OUT), lambda b: (b, 0, 0)),
        out_shape=jax.ShapeDtypeStruct((BATCH, WIN, C_OUT), jnp.float32),
    )
    return call(x, embed_W, embed_b.reshape(1, D_MODEL),
                Wq, bq.reshape(1, D_GE), Wk, bk.reshape(1, D_GE),
                gcn_W, gcn_b.reshape(WIN, 1), mem,
                dec_W1, dec_b1.reshape(1, D_FF),
                dec_W2, dec_b2.reshape(1, C_OUT))
